# Initial kernel scaffold; baseline (speedup 1.0000x reference)
import jax
import jax.numpy as jnp
from jax.experimental import pallas as pl


def kernel(importance):
    raise NotImplementedError("write your pallas kernel here")


# speedup vs baseline: 90.1927x; 90.1927x over previous
import functools

import jax
import jax.numpy as jnp
from jax.experimental import pallas as pl
from jax.experimental.pallas import tpu as pltpu

def _mask_body(k, x_ref, out_ref, key_ref):
    x = x_ref[0]
    bits = jax.lax.bitcast_convert_type(x, jnp.int32)
    key = bits ^ ((bits >> 31) & jnp.int32(0x7FFFFFFF))
    key_ref[...] = key

    c0 = jnp.sum((key >= 0).astype(jnp.int32))
    t0 = jnp.where(c0 >= k, jnp.int32(0), jnp.int32(-2147483648))

    def step(i, t):
        cand = t | (jnp.int32(1) << (jnp.int32(30) - i))
        c = jnp.sum((key_ref[...] >= cand).astype(jnp.int32))
        return jnp.where(c >= k, cand, t)

    t = jax.lax.fori_loop(0, 31, step, t0)
    out_ref[0, 0] = (key_ref[...] >= t).astype(jnp.float32)


def kernel(importance):
    b, h, w = importance.shape
    k = max(1, int(0.75 * h * w))
    mask = pl.pallas_call(
        functools.partial(_mask_body, k),
        grid=(b,),
        in_specs=[pl.BlockSpec((1, h, w), lambda i: (i, 0, 0))],
        out_specs=pl.BlockSpec((1, 1, h, w), lambda i: (i, 0, 0, 0)),
        out_shape=jax.ShapeDtypeStruct((b, 1, h, w), jnp.float32),
        scratch_shapes=[pltpu.VMEM((h, w), jnp.int32)],
    )(importance)
    return (mask, jnp.float32(k / (h * w)))
